# Initial kernel scaffold; baseline (speedup 1.0000x reference)
import jax
import jax.numpy as jnp
from jax.experimental import pallas as pl


def kernel(feat0, feat1, Wq, Wk, Wv, Wm, W1, W2, g1, b1, g2, b2):
    raise NotImplementedError("write your pallas kernel here")


# speedup vs baseline: 1.7518x; 1.7518x over previous
import jax
import jax.numpy as jnp
from jax.experimental import pallas as pl
from jax.experimental.pallas import tpu as pltpu

_LAYER_NAMES = ('self', 'cross', 'self', 'cross')
_D = 256
_H = 8
_DH = _D // _H


def _elu1(x):
    return jnp.where(x > 0, x + 1.0, jnp.exp(jnp.minimum(x, 0.0)))


def _ln(x, g, b, eps=1e-5):
    mu = jnp.mean(x, axis=-1, keepdims=True)
    var = jnp.mean((x - mu) ** 2, axis=-1, keepdims=True)
    return (x - mu) * jax.lax.rsqrt(var + eps) * g + b


def _stats_kernel(src_ref, wk_ref, wv_ref, mask_ref, kv_ref, ks_ref, *, seq_len):
    t = pl.program_id(1)
    s = src_ref[0]
    kf = _elu1(jnp.dot(s, wk_ref[...], preferred_element_type=jnp.float32))
    v = jnp.dot(s, wv_ref[...], preferred_element_type=jnp.float32) * (1.0 / seq_len)
    kv = jax.lax.dot_general(kf, v, (((0,), (0,)), ((), ())),
                             preferred_element_type=jnp.float32)
    kv = kv * mask_ref[...]
    ks = jnp.sum(kf, axis=0, keepdims=True)

    @pl.when(t == 0)
    def _init():
        kv_ref[0] = kv
        ks_ref[0] = ks

    @pl.when(t != 0)
    def _acc():
        kv_ref[0] += kv
        ks_ref[0] += ks


def _apply_kernel(x_ref, kv_ref, ks_ref, mask_ref, wq_ref, wm_ref, w1_ref,
                  w2_ref, g1_ref, b1_ref, g2_ref, b2_ref, y_ref, *, seq_len):
    x = x_ref[0]
    qf = _elu1(jnp.dot(x, wq_ref[...], preferred_element_type=jnp.float32))
    att = jnp.dot(qf, kv_ref[0], preferred_element_type=jnp.float32)
    den = jnp.dot(qf * ks_ref[0], mask_ref[...],
                  preferred_element_type=jnp.float32) + 1e-6
    msg = att * (seq_len / den)
    m = jnp.dot(msg, wm_ref[...], preferred_element_type=jnp.float32)
    m = _ln(m, g1_ref[0], b1_ref[0])
    h = (jnp.dot(x, w1_ref[:_D, :], preferred_element_type=jnp.float32)
         + jnp.dot(m, w1_ref[_D:, :], preferred_element_type=jnp.float32))
    h = jnp.maximum(h, 0.0)
    m2 = jnp.dot(h, w2_ref[...], preferred_element_type=jnp.float32)
    m2 = _ln(m2, g2_ref[0], b2_ref[0])
    y_ref[0] = x + m2


def _encoder(x, src, wq, wk, wv, wm, w1, w2, g1, b1, g2, b2, mask):
    n, seq_len, d = x.shape
    tl = min(512, seq_len)
    nt = seq_len // tl
    from functools import partial

    full2 = lambda shape: pl.BlockSpec(shape, lambda i, t: (0, 0))
    kv, ks = pl.pallas_call(
        partial(_stats_kernel, seq_len=float(seq_len)),
        grid=(n, nt),
        in_specs=[
            pl.BlockSpec((1, tl, d), lambda i, t: (i, t, 0)),
            full2((d, d)),
            full2((d, d)),
            full2((d, d)),
        ],
        out_specs=[
            pl.BlockSpec((1, d, d), lambda i, t: (i, 0, 0)),
            pl.BlockSpec((1, 1, d), lambda i, t: (i, 0, 0)),
        ],
        out_shape=[
            jax.ShapeDtypeStruct((n, d, d), jnp.float32),
            jax.ShapeDtypeStruct((n, 1, d), jnp.float32),
        ],
        compiler_params=pltpu.CompilerParams(
            dimension_semantics=("parallel", "arbitrary")),
    )(src, wk, wv, mask)

    y = pl.pallas_call(
        partial(_apply_kernel, seq_len=float(seq_len)),
        grid=(n, nt),
        in_specs=[
            pl.BlockSpec((1, tl, d), lambda i, t: (i, t, 0)),
            pl.BlockSpec((1, d, d), lambda i, t: (i, 0, 0)),
            pl.BlockSpec((1, 1, d), lambda i, t: (i, 0, 0)),
            full2((d, d)),
            full2((d, d)),
            full2((d, d)),
            full2((2 * d, 2 * d)),
            full2((2 * d, d)),
            pl.BlockSpec((1, d), lambda i, t: (0, 0)),
            pl.BlockSpec((1, d), lambda i, t: (0, 0)),
            pl.BlockSpec((1, d), lambda i, t: (0, 0)),
            pl.BlockSpec((1, d), lambda i, t: (0, 0)),
        ],
        out_specs=pl.BlockSpec((1, tl, d), lambda i, t: (i, t, 0)),
        out_shape=jax.ShapeDtypeStruct((n, seq_len, d), jnp.float32),
        compiler_params=pltpu.CompilerParams(
            dimension_semantics=("parallel", "parallel")),
    )(x, kv, ks, mask, wq, wm, w1, w2, g1, b1, g2, b2)
    return y


def kernel(feat0, feat1, Wq, Wk, Wv, Wm, W1, W2, g1, b1, g2, b2):
    d = feat0.shape[-1]
    ids = jnp.arange(d) // _DH
    mask = (ids[:, None] == ids[None, :]).astype(jnp.float32)
    f0, f1 = feat0, feat1
    for i, name in enumerate(_LAYER_NAMES):
        w = (Wq[i], Wk[i], Wv[i], Wm[i], W1[i], W2[i],
             g1[i].reshape(1, d), b1[i].reshape(1, d),
             g2[i].reshape(1, d), b2[i].reshape(1, d))
        if name == 'self':
            f0 = _encoder(f0, f0, *w, mask)
            f1 = _encoder(f1, f1, *w, mask)
        else:
            f0 = _encoder(f0, f1, *w, mask)
            f1 = _encoder(f1, f0, *w, mask)
    return jnp.concatenate([f0, f1], axis=0)
